# baseline (speedup 1.0000x reference)
import jax
import jax.numpy as jnp
from jax.experimental import pallas as pl
from jax.experimental.pallas import tpu as pltpu

H = 512
RB = 256
NB = H // RB
INV255 = 1.0 / 255.0


def _body_a(tgt_ref, ref_ref, ms_ref, mt_ref, o0_ref, o1_ref):
    ms = ms_ref[...] * INV255
    mt = mt_ref[...] * INV255
    td = jnp.clip((tgt_ref[0] + 1.0) * 0.5, 0.0, 1.0)
    rf = jnp.clip((ref_ref[0] + 1.0) * 0.5, 0.0, 1.0)
    o0_ref[0] = td * mt
    o1_ref[0] = rf * ms


def _body_b(inp_ref, tgt_ref, ms_ref, o2_ref, o3_ref, loss_ref):
    rb = pl.program_id(0)
    c = pl.program_id(1)

    ms = ms_ref[...] * INV255
    td = jnp.clip((tgt_ref[0] + 1.0) * 0.5, 0.0, 1.0)
    idt = jnp.clip(inp_ref[0], 0.0, 1.0)

    o2 = idt * ms
    o3 = td * ms
    o2_ref[0] = o2
    o3_ref[0] = o3

    part = jnp.sum(jnp.abs(o2 - o3))

    @pl.when((rb == 0) & (c == 0))
    def _():
        loss_ref[0] = 0.0

    loss_ref[0] += part


def kernel(input_data, target_data, target_data_eye, mask_src, mask_tar, index, ref):
    del target_data_eye, index
    inp = input_data.reshape(3, H, H)
    tgt = target_data.reshape(3, H, H)
    rf = ref.reshape(3, H, H)
    ms = mask_src.reshape(H, H)
    mt = mask_tar.reshape(H, H)

    img_spec = pl.BlockSpec((1, RB, H), lambda rb, c: (c, rb, 0))
    mask_spec = pl.BlockSpec((RB, H), lambda rb, c: (rb, 0))

    o0, o1 = pl.pallas_call(
        _body_a,
        grid=(NB, 3),
        in_specs=[img_spec, img_spec, mask_spec, mask_spec],
        out_specs=[img_spec, img_spec],
        out_shape=[
            jax.ShapeDtypeStruct((3, H, H), jnp.float32),
            jax.ShapeDtypeStruct((3, H, H), jnp.float32),
        ],
    )(tgt, rf, ms, mt)

    o2, o3, loss = pl.pallas_call(
        _body_b,
        grid=(NB, 3),
        in_specs=[img_spec, img_spec, mask_spec],
        out_specs=[img_spec, img_spec, pl.BlockSpec(memory_space=pltpu.SMEM)],
        out_shape=[
            jax.ShapeDtypeStruct((3, H, H), jnp.float32),
            jax.ShapeDtypeStruct((3, H, H), jnp.float32),
            jax.ShapeDtypeStruct((1,), jnp.float32),
        ],
    )(inp, tgt, ms)

    n = jnp.float32(3 * H * H)
    return (
        o0.reshape(1, 3, H, H),
        o1.reshape(1, 3, H, H),
        o2.reshape(1, 3, H, H),
        o3.reshape(1, 3, H, H),
        loss[0] / n,
    )


# speedup vs baseline: 1.6007x; 1.6007x over previous
import jax
import jax.numpy as jnp
from jax.experimental import pallas as pl
from jax.experimental.pallas import tpu as pltpu

H = 512
RB = 256
NB = H // RB
INV255 = 1.0 / 255.0
INV_N = 1.0 / (3.0 * H * H)


def _body(inp_ref, tgt_ref, ref_ref, ms_ref, mt_ref,
          o0_ref, o1_ref, o2_ref, o3_ref, loss_ref):
    rb = pl.program_id(0)
    c = pl.program_id(1)

    ms = ms_ref[...] * INV255
    mt = mt_ref[...] * INV255

    td = jnp.clip((tgt_ref[0] + 1.0) * 0.5, 0.0, 1.0)
    rf = jnp.clip((ref_ref[0] + 1.0) * 0.5, 0.0, 1.0)
    idt = jnp.clip(inp_ref[0], 0.0, 1.0)

    o0 = td * mt
    o1 = rf * ms
    o2 = idt * ms
    o3 = td * ms

    o0_ref[0] = o0
    o1_ref[0] = o1
    o2_ref[0] = o2
    o3_ref[0] = o3

    part = jnp.sum(jnp.abs(o2 - o3))

    @pl.when((rb == 0) & (c == 0))
    def _():
        loss_ref[0] = 0.0

    loss_ref[0] += part

    @pl.when((rb == NB - 1) & (c == 2))
    def _():
        loss_ref[0] = loss_ref[0] * INV_N


def kernel(input_data, target_data, target_data_eye, mask_src, mask_tar, index, ref):
    del target_data_eye, index
    inp = input_data.reshape(3, H, H)
    tgt = target_data.reshape(3, H, H)
    rf = ref.reshape(3, H, H)
    ms = mask_src.reshape(H, H)
    mt = mask_tar.reshape(H, H)

    img_spec = pl.BlockSpec((1, RB, H), lambda rb, c: (c, rb, 0))
    mask_spec = pl.BlockSpec((RB, H), lambda rb, c: (rb, 0))

    out_shapes = (
        [jax.ShapeDtypeStruct((3, H, H), jnp.float32)] * 4
        + [jax.ShapeDtypeStruct((1,), jnp.float32)]
    )
    out_specs = (
        [img_spec] * 4
        + [pl.BlockSpec(memory_space=pltpu.SMEM)]
    )

    o0, o1, o2, o3, loss = pl.pallas_call(
        _body,
        grid=(NB, 3),
        in_specs=[img_spec, img_spec, img_spec, mask_spec, mask_spec],
        out_specs=out_specs,
        out_shape=out_shapes,
    )(inp, tgt, rf, ms, mt)

    return (
        o0.reshape(1, 3, H, H),
        o1.reshape(1, 3, H, H),
        o2.reshape(1, 3, H, H),
        o3.reshape(1, 3, H, H),
        loss[0],
    )
